# baseline (speedup 1.0000x reference)
import dataclasses
import functools

import jax
import jax.numpy as jnp
from jax import lax
from jax.experimental import pallas as pl
from jax.experimental.pallas import tpu as pltpu
from jax.experimental.pallas import tpu_sc as plsc

N_NODES = 10000
N_PAD = 10240
N_EDGES = 320000
DIM = 128
N_GRAPHS = 64

NC = 2
NS = 16
NW = NC * NS
EPW = N_EDGES // NW
CHUNK = 80
NCHUNKS = EPW // CHUNK
RPS = N_PAD // NS
IB = 3
NBLK = (NCHUNKS + IB - 1) // IB
NSUP = 13
LANES = 16

_MESH = plsc.VectorSubcoreMesh(core_axis_name="c", subcore_axis_name="s",
                               num_cores=NC, num_subcores=NS)

_CP = pltpu.CompilerParams()
if "needs_layout_passes" in pltpu.CompilerParams.__dataclass_fields__:
    _CP = dataclasses.replace(_CP, needs_layout_passes=False)


def _deg_body(dst_hbm, out_hbm, dst_v, deg_v):
    c = lax.axis_index("c")
    s = lax.axis_index("s")
    wid = c * NS + s
    pltpu.sync_copy(dst_hbm.at[wid], dst_v)

    @pl.loop(0, N_PAD // LANES)
    def _(i):
        deg_v[pl.ds(i * LANES, LANES)] = jnp.zeros((LANES,), jnp.float32)

    ones = jnp.ones((LANES,), jnp.float32)

    @pl.loop(0, EPW // LANES)
    def _(i):
        idx = dst_v[pl.ds(i * LANES, LANES)]
        plsc.addupdate_scatter(deg_v, [idx], ones)

    pltpu.sync_copy(deg_v, out_hbm.at[wid])


@functools.partial(
    pl.kernel,
    out_type=jax.ShapeDtypeStruct((NW, N_PAD), jnp.float32),
    mesh=_MESH,
    scratch_types=[
        pltpu.VMEM((EPW,), jnp.int32),
        pltpu.VMEM((N_PAD,), jnp.float32),
    ],
    compiler_params=_CP,
)
def _sc_deg(dst_hbm, out_hbm, dst_v, deg_v):
    _deg_body(dst_hbm, out_hbm, dst_v, deg_v)


def _msg_body(hs_hbm, init_hbm, eidx_hbm, out_hbm,
              ib0, ib1, ib2, rows0, rows1, rows2, acc_sh,
              semb0, semb1, semb2, sem0, sem1, sem2):
    c = lax.axis_index("c")
    s = lax.axis_index("s")
    wid = c * NS + s
    row0 = s * RPS
    pltpu.sync_copy(init_hbm.at[pl.ds(row0, RPS)], acc_sh.at[pl.ds(row0, RPS)])
    plsc.subcore_barrier()

    ibufs = [(ib0, semb0), (ib1, semb1), (ib2, semb2)]
    rbufs = [(rows0, sem0), (rows1, sem1), (rows2, sem2)]

    def bload(b, k):
        pltpu.async_copy(eidx_hbm.at[wid, b], ibufs[k][0], ibufs[k][1])

    def bwait(k):
        pltpu.make_async_copy(eidx_hbm.at[wid, 0], ibufs[k][0],
                              ibufs[k][1]).wait()

    def gather(k, pos, r):
        pltpu.async_copy(hs_hbm.at[ibufs[k][0].at[pos, 0]],
                         rbufs[r][0], rbufs[r][1])

    def gwait(r):
        pltpu.make_async_copy(hs_hbm.at[pl.ds(0, CHUNK)],
                              rbufs[r][0], rbufs[r][1]).wait()

    def scatter(k, pos, r):
        pltpu.sync_copy(rbufs[r][0], acc_sh.at[ibufs[k][0].at[pos, 1]],
                        add=True)

    def slot(p, blk3, do_load, do_gather):
        r = p % 3
        kb = p // 3
        gwait(r)
        scatter(kb, p % 3, r)
        if do_load:
            bload(blk3 + 3 + kb, kb)
        if do_gather:
            kg = ((p + 2) // 3) % 3
            posg = (p + 2) % 3
            if posg == 0:
                bwait(kg)
            gather(kg, posg, (p + 2) % 3)

    for k in range(3):
        bload(k, k)
    bwait(0)
    gather(0, 0, 0)
    gather(0, 1, 1)

    @pl.loop(0, NSUP)
    def _(m):
        blk3 = m * 3
        for p in range(9):
            slot(p, blk3, do_load=(p % 3 == 2), do_gather=True)

    for p in range(8):
        slot(p, 0, do_load=False, do_gather=(p < 6))

    plsc.subcore_barrier()
    pltpu.sync_copy(acc_sh.at[pl.ds(row0, RPS)], out_hbm.at[c, pl.ds(row0, RPS)])


@functools.partial(
    pl.kernel,
    out_type=jax.ShapeDtypeStruct((NC, N_PAD, DIM), jnp.float32),
    mesh=_MESH,
    scratch_types=[
        pltpu.VMEM((IB, 2, CHUNK), jnp.int32),
        pltpu.VMEM((IB, 2, CHUNK), jnp.int32),
        pltpu.VMEM((IB, 2, CHUNK), jnp.int32),
        pltpu.VMEM((CHUNK, DIM), jnp.float32),
        pltpu.VMEM((CHUNK, DIM), jnp.float32),
        pltpu.VMEM((CHUNK, DIM), jnp.float32),
        pltpu.VMEM_SHARED((N_PAD, DIM), jnp.float32),
        pltpu.SemaphoreType.DMA,
        pltpu.SemaphoreType.DMA,
        pltpu.SemaphoreType.DMA,
        pltpu.SemaphoreType.DMA,
        pltpu.SemaphoreType.DMA,
        pltpu.SemaphoreType.DMA,
    ],
)
def _sc_msg(hs_hbm, init_hbm, eidx_hbm, out_hbm,
            ib0, ib1, ib2, rows0, rows1, rows2, acc_sh,
            semb0, semb1, semb2, sem0, sem1, sem2):
    _msg_body(hs_hbm, init_hbm, eidx_hbm, out_hbm,
              ib0, ib1, ib2, rows0, rows1, rows2, acc_sh,
              semb0, semb1, semb2, sem0, sem1, sem2)


def _tc1_body(x_ref, w1_ref, cnt_ref, hs_ref, hsh_ref, dinv_ref):
    h = jnp.dot(x_ref[...], w1_ref[...], preferred_element_type=jnp.float32)
    deg = jnp.sum(cnt_ref[...], axis=0)[:, None] + 1.0
    dinv = lax.rsqrt(deg)
    hs = h * dinv
    hs_ref[...] = hs
    hsh_ref[...] = hs * 0.5
    dinv_ref[...] = dinv


def _tc1(x, w1, cnt):
    return pl.pallas_call(
        _tc1_body,
        out_shape=(
            jax.ShapeDtypeStruct((N_PAD, DIM), jnp.float32),
            jax.ShapeDtypeStruct((N_PAD, DIM), jnp.float32),
            jax.ShapeDtypeStruct((N_PAD, 1), jnp.float32),
        ),
    )(x, w1, cnt)


def _leaky(t):
    return jnp.where(t >= 0.0, t, 0.01 * t)


def _tc2_body(acc_ref, dinv_ref, b1_ref, w2_ref, hs_ref, hsh_ref):
    dinv = dinv_ref[...]
    z = _leaky((acc_ref[0] + acc_ref[1]) * dinv + b1_ref[...])
    h2 = jnp.dot(z, w2_ref[...], preferred_element_type=jnp.float32)
    hs = h2 * dinv
    hs_ref[...] = hs
    hsh_ref[...] = hs * 0.5


def _tc2(acc, dinv, b1, w2):
    return pl.pallas_call(
        _tc2_body,
        out_shape=(
            jax.ShapeDtypeStruct((N_PAD, DIM), jnp.float32),
            jax.ShapeDtypeStruct((N_PAD, DIM), jnp.float32),
        ),
    )(acc, dinv, b1, w2)


def _tc3_body(acc_ref, dinv_ref, b2_ref, batch_ref, fcw_ref, fcb_ref, out_ref):
    acc = acc_ref[0, :N_NODES] + acc_ref[1, :N_NODES]
    z = _leaky(acc * dinv_ref[:N_NODES] + b2_ref[...])
    gids = lax.broadcasted_iota(jnp.int32, (N_GRAPHS, N_NODES), 0)
    m = (batch_ref[...][None, :] == gids).astype(jnp.float32)
    sums = jnp.dot(m, z, preferred_element_type=jnp.float32)
    cnts = jnp.sum(m, axis=1, keepdims=True)
    pooled = sums / jnp.maximum(cnts, 1.0)
    out_ref[...] = (jnp.dot(pooled, fcw_ref[...],
                            preferred_element_type=jnp.float32)
                    + fcb_ref[...])


def _tc3(acc, dinv, b2, batch, fc_w, fc_b):
    return pl.pallas_call(
        _tc3_body,
        out_shape=jax.ShapeDtypeStruct((N_GRAPHS, 1), jnp.float32),
    )(acc, dinv, b2, batch, fc_w, fc_b)


def kernel(x, edge_index, batch, W1, b1, W2, b2, fc_W, fc_b):
    src = edge_index[0]
    dst = edge_index[1]
    src3 = src.reshape(NW, NCHUNKS, CHUNK)
    dst3 = dst.reshape(NW, NCHUNKS, CHUNK)
    dst2 = dst.reshape(NW, EPW)
    eidx = jnp.stack([src3, dst3], axis=2)
    eidx = jnp.concatenate(
        [eidx, jnp.zeros((NW, 1, 2, CHUNK), jnp.int32)], axis=1)
    eidx = eidx.reshape(NW, NBLK, IB, 2, CHUNK)
    x = jnp.concatenate(
        [x, jnp.zeros((N_PAD - N_NODES, DIM), jnp.float32)], axis=0)

    cnt = _sc_deg(dst2)
    hs1, hs1h, dinv = _tc1(x, W1, cnt)
    acc1 = _sc_msg(hs1, hs1h, eidx)
    hs2, hs2h = _tc2(acc1, dinv, b1, W2)
    acc2 = _sc_msg(hs2, hs2h, eidx)
    return _tc3(acc2, dinv, b2, batch, fc_W, fc_b)
